# baseline (speedup 1.0000x reference)
import functools

import jax
import jax.numpy as jnp
from jax import lax
from jax.experimental import pallas as pl
from jax.experimental.pallas import tpu as pltpu
from jax.experimental.pallas import tpu_sc as plsc


def _sc_embed(x, token_table, pos_table, pidx, *, B, L, D, NC, NS, KBUF, GW):
    NW = NC * NS
    seq_per_w = B // NW

    mesh = plsc.VectorSubcoreMesh(core_axis_name="c", subcore_axis_name="s")

    @functools.partial(
        pl.kernel,
        out_type=jax.ShapeDtypeStruct((B, L, D), jnp.float32),
        mesh=mesh,
        scratch_types=[
            pltpu.VMEM((seq_per_w, L), jnp.int32),
            pltpu.VMEM((L,), jnp.int32),
            pltpu.VMEM_SHARED((L, D), jnp.float32),
            *[pltpu.VMEM((L, D), jnp.float32) for _ in range(KBUF)],
            pltpu.SemaphoreType.DMA((KBUF,)),
            pltpu.SemaphoreType.DMA((KBUF,)),
            pltpu.SemaphoreType.DMA((KBUF,)),
        ],
        compiler_params=pltpu.CompilerParams(use_tc_tiling_on_sc=False),
    )
    def body(x_hbm, tok_hbm, pos_hbm, pidx_hbm, out_hbm, idx_vm, pidx_vm,
             pos_sh, *rest):
        bufs = rest[:KBUF]
        sem_g, sem_a, sem_o = rest[KBUF:]
        wid = lax.axis_index("c") * NS + lax.axis_index("s")
        pltpu.sync_copy(x_hbm.at[pl.ds(wid * seq_per_w, seq_per_w)], idx_vm)
        pltpu.sync_copy(pidx_hbm, pidx_vm)
        pltpu.sync_copy(pos_hbm, pos_sh)
        seq0 = wid * seq_per_w

        gh = [None] * KBUF
        ah = [None] * KBUF
        oh = [None] * KBUF
        AW = 1

        for t in range(seq_per_w + GW + AW):
            if t < seq_per_w:
                k = t % KBUF
                if oh[k] is not None:
                    oh[k].wait()
                gh[k] = pltpu.async_copy(
                    tok_hbm.at[idx_vm.at[t]], bufs[k], sem_g.at[k]
                )
            j = t - GW
            if 0 <= j < seq_per_w:
                kj = j % KBUF
                gh[kj].wait()
                ah[kj] = pltpu.async_copy(
                    pos_sh.at[pidx_vm], bufs[kj], sem_a.at[kj], add=True
                )
            i = t - GW - AW
            if 0 <= i < seq_per_w:
                ki = i % KBUF
                ah[ki].wait()
                oh[ki] = pltpu.async_copy(
                    bufs[ki], out_hbm.at[seq0 + i], sem_o.at[ki]
                )
        for k in range(KBUF):
            if oh[k] is not None:
                oh[k].wait()

    return body(x, token_table, pos_table, pidx)


def kernel(x, token_table, pos_table):
    B, L = x.shape
    V, D = token_table.shape

    info = plsc.get_sparse_core_info()
    NC, NS = info.num_cores, info.num_subcores

    pidx = jnp.arange(L, dtype=jnp.int32)

    out = _sc_embed(
        x.astype(jnp.int32), token_table, pos_table, pidx,
        B=B, L=L, D=D, NC=NC, NS=NS, KBUF=8, GW=6,
    )
    return out
